# baseline (speedup 1.0000x reference)
import functools

import jax
import jax.numpy as jnp
from jax import lax
from jax.experimental import pallas as pl
from jax.experimental.pallas import tpu as pltpu
from jax.experimental.pallas import tpu_sc as plsc

_C = 192
_K = 32
_CLAMP = 1.5
_LOW = -1.0
_HIGH = 1.0
_H = 384
_W = 384
_HW = _H * _W

_NC = 1
_NS = 16
_NW = _NC * _NS
_CPW = _C // _NW

_NJ = 48
_CH = 18432
_NCHUNK = _HW // _CH
_NVEC = _CH // 16
_UNROLL = 8

_SCALE = (_K - 1) / (_HIGH - _LOW)
_UJLO = (-_CLAMP - _LOW) * _SCALE + 8.0
_UJHI = (_CLAMP - _LOW) * _SCALE + 8.0

_mesh = plsc.VectorSubcoreMesh(core_axis_name="c", subcore_axis_name="s", num_cores=1)


@functools.partial(
    pl.kernel,
    mesh=_mesh,
    compiler_params=pltpu.CompilerParams(
        needs_layout_passes=False, disable_bounds_checks=True),
    out_type=jax.ShapeDtypeStruct((_C * _HW,), jnp.float32),
    scratch_types=[
        pltpu.VMEM((_CH,), jnp.float32),
        pltpu.VMEM((_CH,), jnp.float32),
        pltpu.VMEM((_CH,), jnp.float32),
        pltpu.VMEM((_CH,), jnp.float32),
        pltpu.VMEM((_C * _NJ,), jnp.int32),
        pltpu.VMEM((_C * _NJ,), jnp.int32),
        pltpu.VMEM((_C,), jnp.float32),
        pltpu.VMEM((_C,), jnp.float32),
        pltpu.VMEM((_C,), jnp.float32),
        pltpu.SemaphoreType.DMA,
        pltpu.SemaphoreType.DMA,
        pltpu.SemaphoreType.DMA,
        pltpu.SemaphoreType.DMA,
    ],
)
def _spline_sc(x_hbm, w01_hbm, w23_hbm, pa_hbm, pb_hbm, pg_hbm,
               out_hbm, xb0, xb1, yb0, yb1, w01v, w23v, pav, pbv, pgv,
               si0, si1, so0, so1):
    wid = lax.axis_index("s") * _NC + lax.axis_index("c")

    pltpu.sync_copy(w01_hbm, w01v)
    pltpu.sync_copy(w23_hbm, w23v)
    pltpu.sync_copy(pa_hbm, pav)
    pltpu.sync_copy(pb_hbm, pbv)
    pltpu.sync_copy(pg_hbm, pgv)

    base = wid * (_CPW * _HW)
    cbase = wid * _CPW
    nchunks = _CPW * _NCHUNK
    xbufs = (xb0, xb1)
    ybufs = (yb0, yb1)
    isems = (si0, si1)
    osems = (so0, so1)

    def start_in(c, b):
        pltpu.async_copy(x_hbm.at[pl.ds(base + c * _CH, _CH)], xbufs[b],
                         isems[b])

    def wait_in(b):
        pltpu.make_async_copy(x_hbm.at[pl.ds(base, _CH)], xbufs[b],
                              isems[b]).wait()

    def start_out(c, b):
        pltpu.async_copy(ybufs[b], out_hbm.at[pl.ds(base + c * _CH, _CH)],
                         osems[b])

    def wait_out(b):
        pltpu.make_async_copy(ybufs[b], out_hbm.at[pl.ds(base, _CH)],
                              osems[b]).wait()

    def compute(c, b):
        chg = cbase + c // _NCHUNK
        ci = jnp.full((16,), chg, jnp.int32)
        pA = plsc.load_gather(pav, [ci])
        pB = plsc.load_gather(pbv, [ci])
        pG = plsc.load_gather(pgv, [ci])
        ibase = chg * _NJ
        xbuf, ybuf = xbufs[b], ybufs[b]

        @plsc.parallel_loop(0, _NVEC, step=1, unroll=_UNROLL)
        def vec_body(i):
            s = pl.ds(i * 16, 16)
            xv = xbuf[s]
            uj = xv * pA + pB
            uj = jnp.maximum(jnp.minimum(uj, _UJHI), _UJLO)
            jv = uj.astype(jnp.int32)
            tf = uj - jv.astype(jnp.float32)
            idx = jv + ibase
            w01 = plsc.load_gather(w01v, [idx])
            w23 = plsc.load_gather(w23v, [idx])
            hi_mask = jnp.int32(-65536)
            g0 = plsc.bitcast(lax.shift_left(w01, 16), jnp.float32)
            g1 = plsc.bitcast(w01 & hi_mask, jnp.float32)
            g2 = plsc.bitcast(lax.shift_left(w23, 16), jnp.float32)
            g3 = plsc.bitcast(w23 & hi_mask, jnp.float32)
            p = ((g3 * tf + g2) * tf + g1) * tf + g0
            ybuf[s] = xv * pG + p

    start_in(0, 0)

    def pair_body(g2, carry):
        c = 2 * g2
        for b in (0, 1):
            cc = c + b
            if b == 0:
                start_in(cc + 1, 1)
            else:
                pl.when(cc + 1 < nchunks)(lambda: start_in(cc + 1, 0))
            wait_in(b)
            pl.when(cc >= 2)(lambda: wait_out(b))
            compute(cc, b)
            start_out(cc, b)
        return carry

    lax.fori_loop(0, nchunks // 2, pair_body, 0)
    wait_out(0)
    wait_out(1)


def kernel(x, a, b, alpha, id_gain, bias):
    f32 = jnp.float32
    iv = jnp.arange(_NJ) - 8

    def tap(d):
        return alpha[:, jnp.clip(iv + d, 0, _K - 1)]

    v0, v1, v2, v3 = tap(-1), tap(0), tap(1), tap(2)
    c0 = (v0 + 4.0 * v1 + v2) / 6.0 + bias[:, None]
    c1 = (v2 - v0) * 0.5
    c2 = (v0 - 2.0 * v1 + v2) * 0.5
    c3 = (3.0 * (v1 - v2) + (v3 - v0)) / 6.0

    def b16(v):
        h = lax.bitcast_convert_type(v.astype(jnp.bfloat16), jnp.uint16)
        return h.astype(jnp.uint32)

    def packpair(lo, hi):
        w = b16(lo) | (b16(hi) << 16)
        return lax.bitcast_convert_type(w, jnp.int32)

    w01 = packpair(c0, c1)
    w23 = packpair(c2, c3)

    pa = (a * _SCALE).astype(f32)
    pb = ((b - _LOW) * _SCALE + 8.0).astype(f32)
    pg = id_gain.astype(f32)

    y = _spline_sc(
        x.reshape(-1).astype(f32),
        w01.reshape(-1), w23.reshape(-1),
        pa, pb, pg,
    )
    return y.reshape(x.shape)


# speedup vs baseline: 1.4601x; 1.4601x over previous
import functools

import jax
import jax.numpy as jnp
from jax import lax
from jax.experimental import pallas as pl
from jax.experimental.pallas import tpu as pltpu
from jax.experimental.pallas import tpu_sc as plsc

_C = 192
_K = 32
_CLAMP = 1.5
_LOW = -1.0
_HIGH = 1.0
_H = 384
_W = 384
_HW = _H * _W

_NC = 2
_NS = 16
_NW = _NC * _NS
_CPW = _C // _NW

_NJ = 48
_RPC = 48
_NCHUNK = _H // _RPC
_NVR = _W // 16
_UNROLL = 2

_SCALE = (_K - 1) / (_HIGH - _LOW)
_UJLO = (-_CLAMP - _LOW) * _SCALE + 8.0
_UJHI = (_CLAMP - _LOW) * _SCALE + 8.0

_mesh = plsc.VectorSubcoreMesh(core_axis_name="c", subcore_axis_name="s")


@functools.partial(
    pl.kernel,
    mesh=_mesh,
    compiler_params=pltpu.CompilerParams(
        needs_layout_passes=False, disable_bounds_checks=True),
    out_type=jax.ShapeDtypeStruct((1, _C, _H, _W), jnp.float32),
    scratch_types=[
        pltpu.VMEM((_RPC, _W), jnp.float32),
        pltpu.VMEM((_RPC, _W), jnp.float32),
        pltpu.VMEM((_RPC, _W), jnp.float32),
        pltpu.VMEM((_RPC, _W), jnp.float32),
        pltpu.VMEM((_C * _NJ,), jnp.int32),
        pltpu.VMEM((_C * _NJ,), jnp.int32),
        pltpu.VMEM((_C,), jnp.float32),
        pltpu.VMEM((_C,), jnp.float32),
        pltpu.VMEM((_C,), jnp.float32),
        pltpu.SemaphoreType.DMA,
        pltpu.SemaphoreType.DMA,
        pltpu.SemaphoreType.DMA,
        pltpu.SemaphoreType.DMA,
    ],
)
def _spline_sc(x_hbm, w01_hbm, w23_hbm, pa_hbm, pb_hbm, pg_hbm,
               out_hbm, xb0, xb1, yb0, yb1, w01v, w23v, pav, pbv, pgv,
               si0, si1, so0, so1):
    wid = lax.axis_index("s") * _NC + lax.axis_index("c")

    pltpu.sync_copy(w01_hbm, w01v)
    pltpu.sync_copy(w23_hbm, w23v)
    pltpu.sync_copy(pa_hbm, pav)
    pltpu.sync_copy(pb_hbm, pbv)
    pltpu.sync_copy(pg_hbm, pgv)

    cbase = wid * _CPW
    nchunks = _CPW * _NCHUNK
    xbufs = (xb0, xb1)
    ybufs = (yb0, yb1)
    isems = (si0, si1)
    osems = (so0, so1)

    def chunk_pos(c):
        chg = cbase + c // _NCHUNK
        row = (c % _NCHUNK) * _RPC
        return chg, row

    def start_in(c, b):
        chg, row = chunk_pos(c)
        pltpu.async_copy(x_hbm.at[0, chg, pl.ds(row, _RPC)], xbufs[b],
                         isems[b])

    def wait_in(b):
        pltpu.make_async_copy(x_hbm.at[0, 0, pl.ds(0, _RPC)], xbufs[b],
                              isems[b]).wait()

    def start_out(c, b):
        chg, row = chunk_pos(c)
        pltpu.async_copy(ybufs[b], out_hbm.at[0, chg, pl.ds(row, _RPC)],
                         osems[b])

    def wait_out(b):
        pltpu.make_async_copy(ybufs[b], out_hbm.at[0, 0, pl.ds(0, _RPC)],
                              osems[b]).wait()

    def compute(c, b):
        chg, _ = chunk_pos(c)
        ci = jnp.full((16,), chg, jnp.int32)
        pA = plsc.load_gather(pav, [ci])
        pB = plsc.load_gather(pbv, [ci])
        pG = plsc.load_gather(pgv, [ci])
        ibase = chg * _NJ
        xbuf, ybuf = xbufs[b], ybufs[b]

        @plsc.parallel_loop(0, _RPC, step=1, unroll=_UNROLL)
        def row_body(r):
            for j in range(_NVR):
                s = pl.ds(j * 16, 16)
                xv = xbuf[r, s]
                uj = xv * pA + pB
                uj = jnp.maximum(jnp.minimum(uj, _UJHI), _UJLO)
                jv = uj.astype(jnp.int32)
                tf = uj - jv.astype(jnp.float32)
                idx = jv + ibase
                w01 = plsc.load_gather(w01v, [idx])
                w23 = plsc.load_gather(w23v, [idx])
                hi_mask = jnp.int32(-65536)
                g0 = plsc.bitcast(lax.shift_left(w01, 16), jnp.float32)
                g1 = plsc.bitcast(w01 & hi_mask, jnp.float32)
                g2 = plsc.bitcast(lax.shift_left(w23, 16), jnp.float32)
                g3 = plsc.bitcast(w23 & hi_mask, jnp.float32)
                p = ((g3 * tf + g2) * tf + g1) * tf + g0
                ybuf[r, s] = xv * pG + p

    start_in(0, 0)

    def pair_body(g2, carry):
        c = 2 * g2
        for b in (0, 1):
            cc = c + b
            if b == 0:
                start_in(cc + 1, 1)
            else:
                pl.when(cc + 1 < nchunks)(lambda: start_in(cc + 1, 0))
            wait_in(b)
            pl.when(cc >= 2)(lambda: wait_out(b))
            compute(cc, b)
            start_out(cc, b)
        return carry

    lax.fori_loop(0, nchunks // 2, pair_body, 0)
    wait_out(0)
    wait_out(1)


def kernel(x, a, b, alpha, id_gain, bias):
    f32 = jnp.float32
    iv = jnp.arange(_NJ) - 8

    def tap(d):
        return alpha[:, jnp.clip(iv + d, 0, _K - 1)]

    v0, v1, v2, v3 = tap(-1), tap(0), tap(1), tap(2)
    c0 = (v0 + 4.0 * v1 + v2) / 6.0 + bias[:, None]
    c1 = (v2 - v0) * 0.5
    c2 = (v0 - 2.0 * v1 + v2) * 0.5
    c3 = (3.0 * (v1 - v2) + (v3 - v0)) / 6.0

    def b16(v):
        h = lax.bitcast_convert_type(v.astype(jnp.bfloat16), jnp.uint16)
        return h.astype(jnp.uint32)

    def packpair(lo, hi):
        w = b16(lo) | (b16(hi) << 16)
        return lax.bitcast_convert_type(w, jnp.int32)

    w01 = packpair(c0, c1)
    w23 = packpair(c2, c3)

    pa = (a * _SCALE).astype(f32)
    pb = ((b - _LOW) * _SCALE + 8.0).astype(f32)
    pg = id_gain.astype(f32)

    return _spline_sc(
        x.astype(f32),
        w01.reshape(-1), w23.reshape(-1),
        pa, pb, pg,
    )


# speedup vs baseline: 2.6696x; 1.8283x over previous
import functools

import jax
import jax.numpy as jnp
from jax import lax
from jax.experimental import pallas as pl
from jax.experimental.pallas import tpu as pltpu
from jax.experimental.pallas import tpu_sc as plsc

_C = 192
_K = 32
_CLAMP = 1.5
_LOW = -1.0
_HIGH = 1.0
_H = 384
_W = 384
_HW = _H * _W

_NC = 2
_NS = 16
_NW = _NC * _NS
_CPW = _C // _NW

_NJ = 48
_RPC = 48
_NCHUNK = _H // _RPC
_NVR = _W // 16
_UNROLL = 8

_SCALE = (_K - 1) / (_HIGH - _LOW)
_UJLO = (-_CLAMP - _LOW) * _SCALE + 8.0
_UJHI = (_CLAMP - _LOW) * _SCALE + 8.0

_mesh = plsc.VectorSubcoreMesh(core_axis_name="c", subcore_axis_name="s")


@functools.partial(
    pl.kernel,
    mesh=_mesh,
    compiler_params=pltpu.CompilerParams(
        needs_layout_passes=False, disable_bounds_checks=True),
    out_type=jax.ShapeDtypeStruct((1, _C, _H, _W), jnp.float32),
    scratch_types=[
        pltpu.VMEM((_RPC, _W), jnp.float32),
        pltpu.VMEM((_RPC, _W), jnp.float32),
        pltpu.VMEM((_RPC, _W), jnp.float32),
        pltpu.VMEM((_RPC, _W), jnp.float32),
        pltpu.VMEM((_C * _NJ,), jnp.int32),
        pltpu.VMEM((_C * _NJ,), jnp.int32),
        pltpu.VMEM((_C,), jnp.float32),
        pltpu.VMEM((_C,), jnp.float32),
        pltpu.VMEM((_C,), jnp.float32),
        pltpu.SemaphoreType.DMA,
        pltpu.SemaphoreType.DMA,
        pltpu.SemaphoreType.DMA,
        pltpu.SemaphoreType.DMA,
    ],
)
def _spline_sc(x_hbm, w01_hbm, w23_hbm, pa_hbm, pb_hbm, pg_hbm,
               out_hbm, xb0, xb1, yb0, yb1, w01v, w23v, pav, pbv, pgv,
               si0, si1, so0, so1):
    wid = lax.axis_index("s") * _NC + lax.axis_index("c")

    pltpu.sync_copy(w01_hbm, w01v)
    pltpu.sync_copy(w23_hbm, w23v)
    pltpu.sync_copy(pa_hbm, pav)
    pltpu.sync_copy(pb_hbm, pbv)
    pltpu.sync_copy(pg_hbm, pgv)

    cbase = wid * _CPW
    nchunks = _CPW * _NCHUNK
    xbufs = (xb0, xb1)
    ybufs = (yb0, yb1)
    isems = (si0, si1)
    osems = (so0, so1)

    def chunk_pos(c):
        chg = cbase + c // _NCHUNK
        row = (c % _NCHUNK) * _RPC
        return chg, row

    def start_in(c, b):
        chg, row = chunk_pos(c)
        pltpu.async_copy(x_hbm.at[0, chg, pl.ds(row, _RPC)], xbufs[b],
                         isems[b])

    def wait_in(b):
        pltpu.make_async_copy(x_hbm.at[0, 0, pl.ds(0, _RPC)], xbufs[b],
                              isems[b]).wait()

    def start_out(c, b):
        chg, row = chunk_pos(c)
        pltpu.async_copy(ybufs[b], out_hbm.at[0, chg, pl.ds(row, _RPC)],
                         osems[b])

    def wait_out(b):
        pltpu.make_async_copy(ybufs[b], out_hbm.at[0, 0, pl.ds(0, _RPC)],
                              osems[b]).wait()

    def compute(c, b):
        chg, _ = chunk_pos(c)
        ci = jnp.full((16,), chg, jnp.int32)
        pA = plsc.load_gather(pav, [ci])
        pB = plsc.load_gather(pbv, [ci])
        pG = plsc.load_gather(pgv, [ci])
        ibase = chg * _NJ
        xbuf, ybuf = xbufs[b], ybufs[b]

        @plsc.parallel_loop(0, _RPC * _NVR, step=1, unroll=_UNROLL)
        def vec_body(i):
            r = i // _NVR
            j = i - r * _NVR
            s = pl.ds(j * 16, 16)
            xv = xbuf[r, s]
            uj = xv * pA + pB
            uj = jnp.maximum(jnp.minimum(uj, _UJHI), _UJLO)
            jv = uj.astype(jnp.int32)
            tf = uj - jv.astype(jnp.float32)
            idx = jv + ibase
            w01 = plsc.load_gather(w01v, [idx])
            w23 = plsc.load_gather(w23v, [idx])
            hi_mask = jnp.int32(-65536)
            g0 = plsc.bitcast(lax.shift_left(w01, 16), jnp.float32)
            g1 = plsc.bitcast(w01 & hi_mask, jnp.float32)
            g2 = plsc.bitcast(lax.shift_left(w23, 16), jnp.float32)
            g3 = plsc.bitcast(w23 & hi_mask, jnp.float32)
            p = ((g3 * tf + g2) * tf + g1) * tf + g0
            ybuf[r, s] = xv * pG + p

    start_in(0, 0)

    def pair_body(g2, carry):
        c = 2 * g2
        for b in (0, 1):
            cc = c + b
            if b == 0:
                start_in(cc + 1, 1)
            else:
                pl.when(cc + 1 < nchunks)(lambda: start_in(cc + 1, 0))
            wait_in(b)
            pl.when(cc >= 2)(lambda: wait_out(b))
            compute(cc, b)
            start_out(cc, b)
        return carry

    lax.fori_loop(0, nchunks // 2, pair_body, 0)
    wait_out(0)
    wait_out(1)


def kernel(x, a, b, alpha, id_gain, bias):
    f32 = jnp.float32
    iv = jnp.arange(_NJ) - 8

    def tap(d):
        return alpha[:, jnp.clip(iv + d, 0, _K - 1)]

    v0, v1, v2, v3 = tap(-1), tap(0), tap(1), tap(2)
    c0 = (v0 + 4.0 * v1 + v2) / 6.0 + bias[:, None]
    c1 = (v2 - v0) * 0.5
    c2 = (v0 - 2.0 * v1 + v2) * 0.5
    c3 = (3.0 * (v1 - v2) + (v3 - v0)) / 6.0

    def b16(v):
        h = lax.bitcast_convert_type(v.astype(jnp.bfloat16), jnp.uint16)
        return h.astype(jnp.uint32)

    def packpair(lo, hi):
        w = b16(lo) | (b16(hi) << 16)
        return lax.bitcast_convert_type(w, jnp.int32)

    w01 = packpair(c0, c1)
    w23 = packpair(c2, c3)

    pa = (a * _SCALE).astype(f32)
    pb = ((b - _LOW) * _SCALE + 8.0).astype(f32)
    pg = id_gain.astype(f32)

    return _spline_sc(
        x.astype(f32),
        w01.reshape(-1), w23.reshape(-1),
        pa, pb, pg,
    )


# speedup vs baseline: 2.7899x; 1.0451x over previous
import functools

import jax
import jax.numpy as jnp
from jax import lax
from jax.experimental import pallas as pl
from jax.experimental.pallas import tpu as pltpu
from jax.experimental.pallas import tpu_sc as plsc

_C = 192
_K = 32
_CLAMP = 1.5
_LOW = -1.0
_HIGH = 1.0
_H = 384
_W = 384
_HW = _H * _W

_NC = 2
_NS = 16
_NW = _NC * _NS
_CPW = _C // _NW

_NJ = 48
_RPC = 48
_NCHUNK = _H // _RPC
_NVR = _W // 16
_UNROLL = 8

_SCALE = (_K - 1) / (_HIGH - _LOW)
_UJLO = (-_CLAMP - _LOW) * _SCALE + 8.0
_UJHI = (_CLAMP - _LOW) * _SCALE + 8.0

_mesh = plsc.VectorSubcoreMesh(core_axis_name="c", subcore_axis_name="s")


@functools.partial(
    pl.kernel,
    mesh=_mesh,
    compiler_params=pltpu.CompilerParams(
        needs_layout_passes=False, disable_bounds_checks=True),
    out_type=jax.ShapeDtypeStruct((1, _C, _H, _W), jnp.float32),
    scratch_types=[
        pltpu.VMEM((_RPC, _W), jnp.float32),
        pltpu.VMEM((_RPC, _W), jnp.float32),
        pltpu.VMEM((_RPC, _W), jnp.float32),
        pltpu.VMEM((_RPC, _W), jnp.float32),
        pltpu.VMEM((_C * _NJ,), jnp.int32),
        pltpu.VMEM((_C * _NJ,), jnp.int32),
        pltpu.VMEM((5 * _C,), jnp.float32),
        pltpu.SemaphoreType.DMA,
        pltpu.SemaphoreType.DMA,
        pltpu.SemaphoreType.DMA,
        pltpu.SemaphoreType.DMA,
    ],
)
def _spline_sc(x_hbm, w01_hbm, w23_hbm, pp_hbm,
               out_hbm, xb0, xb1, yb0, yb1, w01v, w23v, ppv,
               si0, si1, so0, so1):
    wid = lax.axis_index("s") * _NC + lax.axis_index("c")
    cbase = wid * _CPW
    nchunks = _CPW * _NCHUNK
    xbufs = (xb0, xb1)
    ybufs = (yb0, yb1)
    isems = (si0, si1)
    osems = (so0, so1)

    def chunk_pos(c):
        chg = cbase + c // _NCHUNK
        row = (c % _NCHUNK) * _RPC
        return chg, row

    def start_in(c, b):
        chg, row = chunk_pos(c)
        pltpu.async_copy(x_hbm.at[0, chg, pl.ds(row, _RPC)], xbufs[b],
                         isems[b])

    def wait_in(b):
        pltpu.make_async_copy(x_hbm.at[0, 0, pl.ds(0, _RPC)], xbufs[b],
                              isems[b]).wait()

    def start_out(c, b):
        chg, row = chunk_pos(c)
        pltpu.async_copy(ybufs[b], out_hbm.at[0, chg, pl.ds(row, _RPC)],
                         osems[b])

    def wait_out(b):
        pltpu.make_async_copy(ybufs[b], out_hbm.at[0, 0, pl.ds(0, _RPC)],
                              osems[b]).wait()

    def compute(c, b):
        chg, _ = chunk_pos(c)
        ci = jnp.full((16,), chg, jnp.int32)
        pA = plsc.load_gather(ppv, [ci])
        pB = plsc.load_gather(ppv, [ci + _C])
        pLO = plsc.load_gather(ppv, [ci + 2 * _C])
        pHI = plsc.load_gather(ppv, [ci + 3 * _C])
        pG = plsc.load_gather(ppv, [ci + 4 * _C])
        xbuf, ybuf = xbufs[b], ybufs[b]

        @plsc.parallel_loop(0, _RPC * _NVR, step=1, unroll=_UNROLL)
        def vec_body(i):
            r = i // _NVR
            j = i - r * _NVR
            s = pl.ds(j * 16, 16)
            xv = xbuf[r, s]
            uj = xv * pA + pB
            uj = jnp.maximum(jnp.minimum(uj, pHI), pLO)
            jv = uj.astype(jnp.int32)
            tf = uj - jv.astype(jnp.float32)
            w01 = plsc.load_gather(w01v, [jv])
            w23 = plsc.load_gather(w23v, [jv])
            hi_mask = jnp.int32(-65536)
            g0 = plsc.bitcast(lax.shift_left(w01, 16), jnp.float32)
            g1 = plsc.bitcast(w01 & hi_mask, jnp.float32)
            g2 = plsc.bitcast(lax.shift_left(w23, 16), jnp.float32)
            g3 = plsc.bitcast(w23 & hi_mask, jnp.float32)
            p = ((g3 * tf + g2) * tf + g1) * tf + g0
            ybuf[r, s] = xv * pG + p

    start_in(0, 0)
    start_in(1, 1)
    pltpu.sync_copy(w01_hbm, w01v)
    pltpu.sync_copy(w23_hbm, w23v)
    pltpu.sync_copy(pp_hbm, ppv)

    def pair_body(g2, carry):
        c = 2 * g2
        for b in (0, 1):
            cc = c + b
            wait_in(b)
            pl.when(cc >= 2)(lambda: wait_out(b))
            compute(cc, b)
            start_out(cc, b)
            pl.when(cc + 2 < nchunks)(lambda: start_in(cc + 2, b))
        return carry

    lax.fori_loop(0, nchunks // 2, pair_body, 0)
    wait_out(0)
    wait_out(1)


def kernel(x, a, b, alpha, id_gain, bias):
    f32 = jnp.float32
    iv = jnp.arange(_NJ) - 8

    def tap(d):
        return alpha[:, jnp.clip(iv + d, 0, _K - 1)]

    v0, v1, v2, v3 = tap(-1), tap(0), tap(1), tap(2)
    c0 = (v0 + 4.0 * v1 + v2) / 6.0 + bias[:, None]
    c1 = (v2 - v0) * 0.5
    c2 = (v0 - 2.0 * v1 + v2) * 0.5
    c3 = (3.0 * (v1 - v2) + (v3 - v0)) / 6.0

    def b16(v):
        h = lax.bitcast_convert_type(v.astype(jnp.bfloat16), jnp.uint16)
        return h.astype(jnp.uint32)

    def packpair(lo, hi):
        w = b16(lo) | (b16(hi) << 16)
        return lax.bitcast_convert_type(w, jnp.int32)

    w01 = packpair(c0, c1)
    w23 = packpair(c2, c3)

    cb = jnp.arange(_C, dtype=f32) * _NJ
    pa = (a * _SCALE).astype(f32)
    pb = ((b - _LOW) * _SCALE + 8.0).astype(f32) + cb
    plo = cb + f32(_UJLO)
    phi = cb + f32(_UJHI)
    pg = id_gain.astype(f32)
    pp = jnp.concatenate([pa, pb, plo, phi, pg])

    return _spline_sc(
        x.astype(f32),
        w01.reshape(-1), w23.reshape(-1),
        pp,
    )
